# baseline (speedup 1.0000x reference)
import jax
import jax.numpy as jnp
import numpy as np
from jax.experimental import pallas as pl
from jax.experimental.pallas import tpu as pltpu

NV, NE, NF = 4096, 8192, 4096
BM = 1024
BK = 2048
NSLOT = 4

_G_SHAPES = [(NV, NV), (NE, NV), (NE, NE), (NF, NE), (NF, NF)]
_X_OFF = [0, 0, NV, NV, NV + NE]
_OUT_ID = [0, 1, 1, 2, 2]
_OUT_ADD = [0, 0, 1, 0, 1]


def _build_schedule():
    cols = {k: [] for k in
            ('seg', 'roff', 'coff', 'blk', 'xrow', 'firstk',
             'lastk', 'outid', 'oblk', 'oadd')}
    blk_base = 0
    for g, (m, kdim) in enumerate(_G_SHAPES):
        n_i, n_k = m // BM, kdim // BK
        for i in range(n_i):
            for k in range(n_k):
                cols['seg'].append(g)
                cols['roff'].append(i * BM)
                cols['coff'].append(k * BK)
                cols['blk'].append(blk_base + i)
                cols['xrow'].append(_X_OFF[g] + k * BK)
                cols['firstk'].append(1 if k == 0 else 0)
                cols['lastk'].append(1 if k == n_k - 1 else 0)
                cols['outid'].append(_OUT_ID[g])
                cols['oblk'].append(i)
                cols['oadd'].append(_OUT_ADD[g])
        blk_base += n_i
    order = ['seg', 'roff', 'coff', 'blk', 'xrow', 'firstk',
             'lastk', 'outid', 'oblk', 'oadd']
    return [np.asarray(cols[k], dtype=np.int32) for k in order], blk_base


_SCHED, _NBLK = _build_schedule()
_NSTEP = len(_SCHED[0])


def _big_kernel(seg_ref, roff_ref, coff_ref, blk_ref, xrow_ref, fk_ref,
                lk_ref, oid_ref, oblk_ref, oadd_ref,
                xall_ref, wstack_ref, g0_ref, g1_ref, g2_ref, g3_ref, g4_ref,
                ov_ref, oe_ref, of_ref, t_ref, buf_ref, sem_ref):
    s = pl.program_id(0)
    g_refs = [g0_ref, g1_ref, g2_ref, g3_ref, g4_ref]
    o_refs = [ov_ref, oe_ref, of_ref]

    def enqueue(t, slot):
        half = BM // 2
        for c in range(5):
            @pl.when(seg_ref[t] == c)
            def _(c=c):
                r0 = pl.multiple_of(roff_ref[t], BM)
                c0 = pl.multiple_of(coff_ref[t], BK)
                src_lo = g_refs[c].at[pl.ds(r0, half), pl.ds(c0, BK)]
                src_hi = g_refs[c].at[pl.ds(r0 + half, half), pl.ds(c0, BK)]
                pltpu.make_async_copy(
                    src_lo, buf_ref.at[slot, pl.ds(0, half)],
                    sem_ref.at[slot]).start()
                pltpu.make_async_copy(
                    src_hi, buf_ref.at[slot, pl.ds(half, half)],
                    sem_ref.at[slot]).start()

    @pl.when(s == 0)
    def _():
        for j in range(NSLOT):
            enqueue(j, j)

    slot = jax.lax.rem(s, NSLOT)
    half = BM // 2
    for h in range(2):
        pltpu.make_async_copy(
            g0_ref.at[pl.ds(h * half, half), pl.ds(0, BK)],
            buf_ref.at[slot, pl.ds(h * half, half)],
            sem_ref.at[slot]).wait()

    g16 = buf_ref[slot].astype(jnp.bfloat16)
    x_blk = xall_ref[pl.ds(pl.multiple_of(xrow_ref[s], BK), BK), :]
    part = jax.lax.dot_general(
        x_blk, g16,
        dimension_numbers=(((0,), (1,)), ((), ())),
        preferred_element_type=jnp.float32)

    acc = jnp.where(fk_ref[s] == 1, jnp.zeros_like(part), t_ref[...]) + part
    t_ref[...] = acc

    @pl.when(lk_ref[s] == 1)
    def _():
        w16 = wstack_ref[seg_ref[s]].astype(jnp.bfloat16)
        contrib = jax.lax.dot_general(
            w16, acc.astype(jnp.bfloat16),
            dimension_numbers=(((0,), (0,)), ((), ())),
            preferred_element_type=jnp.float32)
        j = oblk_ref[s]
        for oid in range(3):
            @pl.when(oid_ref[s] == oid)
            def _(oid=oid):
                o_ref = o_refs[oid]
                prev = jnp.where(oadd_ref[s] == 1, o_ref[j],
                                 jnp.zeros_like(contrib))
                o_ref[j] = prev + contrib

    @pl.when(s + NSLOT < _NSTEP)
    def _():
        enqueue(s + NSLOT, slot)


@jax.jit
def kernel(xv, xe, xf, Gv2v, Gv2e, Ge2e, Ge2f, Gf2f, Wv2v, Wve, Wee, Wef, Wff):
    xall = jnp.concatenate([xv, xe, xf], axis=0).astype(jnp.bfloat16)
    wstack = jnp.stack([Wv2v, Wve, Wee, Wef, Wff], axis=0)

    hbm_spec = pl.BlockSpec(memory_space=pltpu.MemorySpace.HBM)
    ovt, oet, oft = pl.pallas_call(
        _big_kernel,
        grid_spec=pltpu.PrefetchScalarGridSpec(
            num_scalar_prefetch=10,
            grid=(_NSTEP,),
            in_specs=[
                pl.BlockSpec((NV + NE + NF, 32), lambda s, *_: (0, 0)),
                pl.BlockSpec((5, 32, 32), lambda s, *_: (0, 0, 0)),
                hbm_spec, hbm_spec, hbm_spec, hbm_spec, hbm_spec,
            ],
            out_specs=(
                pl.BlockSpec((NV // BM, 32, BM), lambda s, *_: (0, 0, 0)),
                pl.BlockSpec((NE // BM, 32, BM), lambda s, *_: (0, 0, 0)),
                pl.BlockSpec((NF // BM, 32, BM), lambda s, *_: (0, 0, 0)),
            ),
            scratch_shapes=[
                pltpu.VMEM((32, BM), jnp.float32),
                pltpu.VMEM((NSLOT, BM, BK), jnp.float32),
                pltpu.SemaphoreType.DMA((NSLOT,)),
            ],
        ),
        out_shape=(
            jax.ShapeDtypeStruct((NV // BM, 32, BM), jnp.float32),
            jax.ShapeDtypeStruct((NE // BM, 32, BM), jnp.float32),
            jax.ShapeDtypeStruct((NF // BM, 32, BM), jnp.float32),
        ),
        compiler_params=pltpu.CompilerParams(
            dimension_semantics=("arbitrary",),
        ),
    )(*_SCHED, xall, wstack, Gv2v, Gv2e, Ge2e, Ge2f, Gf2f)

    zv = ovt.transpose(0, 2, 1).reshape(NV, 32)
    ze = oet.transpose(0, 2, 1).reshape(NE, 32)
    zf = oft.transpose(0, 2, 1).reshape(NF, 32)
    return (zv, ze, zf)
